# baseline (speedup 1.0000x reference)
import functools

import jax
import jax.numpy as jnp
from jax import lax
from jax.experimental import pallas as pl
from jax.experimental.pallas import tpu as pltpu
from jax.experimental.pallas import tpu_sc as plsc

_NC = 2
_NS = 16
_NW = _NC * _NS


def _gather_rows(char_hash_table, token_flat, embed_weight):
    B = token_flat.shape[0]
    D = embed_weight.shape[1]
    b_per_w = B // _NW
    ch = 512
    n_chunks = b_per_w // ch
    mesh = plsc.VectorSubcoreMesh(core_axis_name="c", subcore_axis_name="s")

    @functools.partial(
        pl.kernel,
        out_type=jax.ShapeDtypeStruct((B, D), jnp.float32),
        mesh=mesh,
        scratch_types=[
            pltpu.VMEM((b_per_w,), jnp.int32),
            pltpu.VMEM((b_per_w,), jnp.int32),
            pltpu.VMEM((ch, D), jnp.float32),
            pltpu.SemaphoreType.DMA,
        ],
    )
    def sc_kernel(hash_hbm, tok_hbm, emb_hbm, out_hbm, tok_v, idx_v, rows_v, sem):
        wid = lax.axis_index("s") * _NC + lax.axis_index("c")
        base = wid * b_per_w
        pltpu.sync_copy(tok_hbm.at[pl.ds(base, b_per_w)], tok_v)
        pltpu.async_copy(hash_hbm.at[tok_v], idx_v, sem).wait()
        for i in range(n_chunks):
            pltpu.async_copy(
                emb_hbm.at[idx_v.at[pl.ds(i * ch, ch)]], rows_v, sem
            ).wait()
            pltpu.sync_copy(rows_v, out_hbm.at[pl.ds(base + i * ch, ch)])

    return sc_kernel(char_hash_table, token_flat, embed_weight)


def _project(rows, proj_t, scale):
    M, K = rows.shape
    N = proj_t.shape[1]
    bm = 1024

    def body(s_ref, x_ref, w_ref, o_ref):
        x = x_ref[...].astype(jnp.bfloat16)
        w = w_ref[...].astype(jnp.bfloat16)
        acc = jnp.dot(x, w, preferred_element_type=jnp.float32)
        o_ref[...] = acc * s_ref[0, 0]

    return pl.pallas_call(
        body,
        grid=(M // bm,),
        in_specs=[
            pl.BlockSpec(memory_space=pltpu.SMEM),
            pl.BlockSpec((bm, K), lambda i: (i, 0)),
            pl.BlockSpec((K, N), lambda i: (0, 0)),
        ],
        out_specs=pl.BlockSpec((bm, N), lambda i: (i, 0)),
        out_shape=jax.ShapeDtypeStruct((M, N), jnp.float32),
    )(scale.reshape(1, 1), rows, proj_t)


def kernel(token_ids, embed_weight, proj_weight, scale, char_hash_table):
    b, s = token_ids.shape
    tok_flat = token_ids.reshape(-1)
    rows = _gather_rows(char_hash_table, tok_flat, embed_weight)
    proj_t = proj_weight.T
    out = _project(rows, proj_t, scale)
    return out.reshape(b, s, proj_weight.shape[0])
